# baseline (speedup 1.0000x reference)
import functools

import numpy as np
import jax
import jax.numpy as jnp
from jax import lax
from jax.experimental import pallas as pl
from jax.experimental.pallas import tpu as pltpu
from jax.experimental.pallas import tpu_sc as plsc

R = 128
V = 100000
K = 50
CAP = 512
CHUNK = 10000
NCHUNK = V // CHUNK
CBLK = 2048
NCB = (V + CBLK - 1) // CBLK
LAST_W = V - (NCB - 1) * CBLK
INT_MIN = np.int32(-2147483648)
NEG_INF = np.float32(-np.inf)
TINY = np.float32(1.1754943508222875e-38)


def _monokey(x):
    b = lax.bitcast_convert_type(x, jnp.int32)
    return jnp.where(b >= 0, b, b ^ jnp.int32(0x7FFFFFFF))


def _inv_monokey(k):
    b = jnp.where(k >= 0, k, k ^ jnp.int32(0x7FFFFFFF))
    return lax.bitcast_convert_type(b, jnp.float32)


def _kth_largest_key(keys, k):
    nonneg = jnp.sum(jnp.where(keys >= 0, 1, 0).astype(jnp.int32), axis=1,
                     keepdims=True)
    t = jnp.where(nonneg >= k, jnp.int32(0), INT_MIN)

    def body(i, t):
        cand = t | (jnp.int32(1) << (jnp.int32(30) - i))
        cnt = jnp.sum(jnp.where(keys >= cand, 1, 0).astype(jnp.int32),
                      axis=1, keepdims=True)
        return jnp.where(cnt >= k, cand, t)

    return lax.fori_loop(0, 31, body, t)


def _stage_a_kernel(x_ref, t0_ref, acc_ref):
    j = pl.program_id(1)
    limit = jnp.where(j == NCB - 1, jnp.int32(LAST_W), jnp.int32(CBLK))
    col = lax.broadcasted_iota(jnp.int32, (8, CBLK), 1)
    x = jnp.where(col < limit, x_ref[...], NEG_INF)

    m = x[:, 0:128]
    for s in range(1, CBLK // 128):
        m = jnp.maximum(m, x[:, s * 128:(s + 1) * 128])

    @pl.when(j == 0)
    def _():
        acc_ref[...] = m

    @pl.when(j > 0)
    def _():
        acc_ref[...] = jnp.maximum(acc_ref[...], m)

    @pl.when(j == NCB - 1)
    def _():
        keys = _monokey(acc_ref[...])
        t = _kth_largest_key(keys, K)
        t0 = _inv_monokey(t - 4)
        t0_ref[...] = jnp.broadcast_to(t0, (8, 16))


def _stage_a(logits):
    return pl.pallas_call(
        _stage_a_kernel,
        grid=(R // 8, NCB),
        in_specs=[pl.BlockSpec((8, CBLK), lambda i, j: (i, j))],
        out_specs=pl.BlockSpec((8, 16), lambda i, j: (i, 0)),
        out_shape=jax.ShapeDtypeStruct((R, 16), jnp.float32),
        scratch_shapes=[pltpu.VMEM((8, 128), jnp.float32)],
    )(logits)


@functools.cache
def _build_stage_b():
    mesh = plsc.VectorSubcoreMesh(core_axis_name="c", subcore_axis_name="s")
    return functools.partial(
        pl.kernel,
        mesh=mesh,
        compiler_params=pltpu.CompilerParams(needs_layout_passes=False),
        out_type=[
            jax.ShapeDtypeStruct((R * CAP,), jnp.int32),
            jax.ShapeDtypeStruct((R * CAP,), jnp.float32),
            jax.ShapeDtypeStruct((R * 16,), jnp.int32),
        ],
        scratch_types=[
            pltpu.VMEM((4 * CHUNK,), jnp.float32),
            pltpu.VMEM((4 * CHUNK,), jnp.float32),
            pltpu.VMEM((4 * CAP,), jnp.int32),
            pltpu.VMEM((4 * CAP,), jnp.float32),
            pltpu.VMEM((64,), jnp.float32),
            pltpu.VMEM((64,), jnp.int32),
            pltpu.SemaphoreType.DMA,
            pltpu.SemaphoreType.DMA,
            pltpu.SemaphoreType.DMA,
        ],
    )(_stage_b_body)


def _stage_b_body(flat_hbm, t0_hbm, idx_hbm, val_hbm, cnt_hbm,
                  buf0, buf1, idx_v, val_v, t0_v, cnt_v, sem0, sem1, semg):
    wid = lax.axis_index("s") * 2 + lax.axis_index("c")
    row0 = wid * 4
    iota = lax.iota(jnp.int32, 16)

    pltpu.sync_copy(t0_hbm.at[pl.ds(pl.multiple_of(row0 * 16, 8), 64)], t0_v)
    tvecs = [t0_v[pl.ds(rl * 16, 16)] for rl in range(4)]
    idx_rows = [idx_v.at[pl.ds(rl * CAP, CAP)] for rl in range(4)]

    def zbody(z, _):
        idx_v[pl.ds(z * 16, 16)] = jnp.zeros((16,), jnp.int32)
        return 0
    lax.fori_loop(0, 4 * CAP // 16, zbody, 0)

    def issue(buf, sem, c):
        for rl in range(4):
            src = pl.multiple_of((row0 + rl) * V + c * CHUNK, 8)
            pltpu.async_copy(flat_hbm.at[pl.ds(src, CHUNK)],
                             buf.at[pl.ds(rl * CHUNK, CHUNK)], sem)

    def drain(buf, sem):
        for rl in range(4):
            pltpu.make_async_copy(flat_hbm.at[pl.ds(0, CHUNK)],
                                  buf.at[pl.ds(rl * CHUNK, CHUNK)], sem).wait()

    def scan_buf(buf, c, offs):
        def inner(v, offs):
            new = []
            for rl in range(4):
                x = buf[pl.ds(rl * CHUNK + v * 16, 16)]
                mask = x >= tvecs[rl]
                mi = jnp.where(mask, 1, 0).astype(jnp.int32)
                cum = plsc.cumsum(mi)
                pc = plsc.all_reduce_population_count(mask)
                idxv = iota + ((row0 + rl) * V + c * CHUNK + v * 16)
                tgt = offs[rl] + cum - 1
                smask = jnp.logical_and(mask, tgt < CAP)
                plsc.store_scatter(idx_rows[rl], [tgt], idxv, mask=smask)
                new.append(jnp.minimum(offs[rl] + pc, CAP))
            return tuple(new)

        return lax.fori_loop(0, CHUNK // 16, inner, offs)

    issue(buf0, sem0, 0)
    offs = (jnp.zeros((16,), jnp.int32),) * 4

    def pair(c2, offs):
        c0 = c2 * 2
        issue(buf1, sem1, c0 + 1)
        drain(buf0, sem0)
        offs = scan_buf(buf0, c0, offs)

        @pl.when(c2 < NCHUNK // 2 - 1)
        def _():
            issue(buf0, sem0, c0 + 2)

        drain(buf1, sem1)
        offs = scan_buf(buf1, c0 + 1, offs)
        return offs

    offs = lax.fori_loop(0, NCHUNK // 2, pair, offs)

    for rl in range(4):
        cnt_v[pl.ds(rl * 16, 16)] = offs[rl]
        for g in range(CAP // 128):
            pltpu.async_copy(
                flat_hbm.at[idx_v.at[pl.ds(rl * CAP + g * 128, 128)]],
                val_v.at[pl.ds(rl * CAP + g * 128, 128)], semg).wait()

    pltpu.sync_copy(idx_v, idx_hbm.at[pl.ds(pl.multiple_of(row0 * CAP, 8),
                                            4 * CAP)])
    pltpu.sync_copy(val_v, val_hbm.at[pl.ds(pl.multiple_of(row0 * CAP, 8),
                                            4 * CAP)])
    pltpu.sync_copy(cnt_v, cnt_hbm.at[pl.ds(pl.multiple_of(row0 * 16, 8), 64)])


def _threefry_bits(x1u):
    k0 = jnp.uint32(0)
    k1 = jnp.uint32(42)
    k2 = k0 ^ k1 ^ jnp.uint32(0x1BD11BDA)
    ks = (k0, k1, k2)
    x0 = jnp.zeros_like(x1u) + k0
    x1 = x1u + k1
    rots = ((13, 15, 26, 6), (17, 29, 16, 24))
    for i in range(5):
        for rr in rots[i % 2]:
            x0 = x0 + x1
            x1 = (x1 << rr) | (x1 >> (32 - rr))
            x1 = x1 ^ x0
        x0 = x0 + ks[(i + 1) % 3]
        x1 = x1 + ks[(i + 2) % 3] + jnp.uint32(i + 1)
    return x0 ^ x1


def _stage_c_kernel(val_ref, idx_ref, cnt_ref, temp_ref, out_ref):
    vals = val_ref[...]
    idx = idx_ref[...]
    cnt = cnt_ref[...][:, 0:1]
    temp = temp_ref[...]

    colj = lax.broadcasted_iota(jnp.int32, (R, CAP), 1)
    rowi = lax.broadcasted_iota(jnp.int32, (R, CAP), 0)
    valid = colj < cnt

    l = vals / temp
    keys = jnp.where(valid, _monokey(l), INT_MIN)
    kth = _kth_largest_key(keys, K)
    keep = keys >= kth

    m = jnp.max(jnp.where(keep, l, NEG_INF), axis=1, keepdims=True)
    e = jnp.where(keep, jnp.exp(l - m), jnp.float32(0.0))
    s = jnp.sum(e, axis=1, keepdims=True)
    lp = jnp.log(e / s + jnp.float32(1e-30))

    bits = _threefry_bits(idx.astype(jnp.uint32))
    f = lax.bitcast_convert_type(
        (bits >> jnp.uint32(9)) | jnp.uint32(0x3F800000),
        jnp.float32) - jnp.float32(1.0)
    u = jnp.maximum(TINY, f + TINY)
    g = -jnp.log(-jnp.log(u))

    score = jnp.where(keep, lp + g, NEG_INF)
    best = jnp.max(score, axis=1, keepdims=True)
    wincol = jnp.min(jnp.where(score == best, colj, jnp.int32(2 ** 30)),
                     axis=1, keepdims=True)
    vocab_idx = idx - rowi * jnp.int32(V)
    token = jnp.sum(jnp.where(colj == wincol, vocab_idx, 0), axis=1,
                    keepdims=True)
    out_ref[...] = token


def _stage_c(vals, idx, cnts, temps):
    return pl.pallas_call(
        _stage_c_kernel,
        in_specs=[pl.BlockSpec((R, CAP), lambda: (0, 0)),
                  pl.BlockSpec((R, CAP), lambda: (0, 0)),
                  pl.BlockSpec((R, 16), lambda: (0, 0)),
                  pl.BlockSpec((R, 1), lambda: (0, 0))],
        out_specs=pl.BlockSpec((R, 1), lambda: (0, 0)),
        out_shape=jax.ShapeDtypeStruct((R, 1), jnp.int32),
    )(vals, idx, cnts, temps)


def kernel(logits, temperatures, top_k=50):
    del top_k
    logits = logits.astype(jnp.float32)
    t0 = _stage_a(logits)
    idx, vals, cnts = _build_stage_b()(logits.reshape(-1), t0.reshape(-1))
    tok = _stage_c(vals.reshape(R, CAP), idx.reshape(R, CAP),
                   cnts.reshape(R, 16),
                   temperatures.astype(jnp.float32).reshape(R, 1))
    return tok.reshape(R)


# speedup vs baseline: 1.3416x; 1.3416x over previous
import functools

import numpy as np
import jax
import jax.numpy as jnp
from jax import lax
from jax.experimental import pallas as pl
from jax.experimental.pallas import tpu as pltpu
from jax.experimental.pallas import tpu_sc as plsc

R = 128
V = 100000
K = 50
CAP = 512
CHUNK = 10000
NCHUNK = V // CHUNK
CBLK = 2048
NCB = (V + CBLK - 1) // CBLK
LAST_W = V - (NCB - 1) * CBLK
INT_MIN = np.int32(-2147483648)
NEG_INF = np.float32(-np.inf)
TINY = np.float32(1.1754943508222875e-38)


def _monokey(x):
    b = lax.bitcast_convert_type(x, jnp.int32)
    return jnp.where(b >= 0, b, b ^ jnp.int32(0x7FFFFFFF))


def _inv_monokey(k):
    b = jnp.where(k >= 0, k, k ^ jnp.int32(0x7FFFFFFF))
    return lax.bitcast_convert_type(b, jnp.float32)


def _kth_largest_key(keys, k):
    nonneg = jnp.sum(jnp.where(keys >= 0, 1, 0).astype(jnp.int32), axis=1,
                     keepdims=True)
    t = jnp.where(nonneg >= k, jnp.int32(0), INT_MIN)

    def body(i, t):
        cand = t | (jnp.int32(1) << (jnp.int32(30) - i))
        cnt = jnp.sum(jnp.where(keys >= cand, 1, 0).astype(jnp.int32),
                      axis=1, keepdims=True)
        return jnp.where(cnt >= k, cand, t)

    return lax.fori_loop(0, 31, body, t)


def _stage_a_kernel(x_ref, t0_ref, acc_ref):
    j = pl.program_id(1)
    limit = jnp.where(j == NCB - 1, jnp.int32(LAST_W), jnp.int32(CBLK))
    col = lax.broadcasted_iota(jnp.int32, (8, CBLK), 1)
    x = jnp.where(col < limit, x_ref[...], NEG_INF)

    m = x[:, 0:128]
    for s in range(1, CBLK // 128):
        m = jnp.maximum(m, x[:, s * 128:(s + 1) * 128])

    @pl.when(j == 0)
    def _():
        acc_ref[...] = m

    @pl.when(j > 0)
    def _():
        acc_ref[...] = jnp.maximum(acc_ref[...], m)

    @pl.when(j == NCB - 1)
    def _():
        keys = _monokey(acc_ref[...])
        t = _kth_largest_key(keys, K)
        t0 = _inv_monokey(t - 4)
        t0_ref[...] = jnp.broadcast_to(t0, (8, 16))


def _stage_a(logits):
    return pl.pallas_call(
        _stage_a_kernel,
        grid=(R // 8, NCB),
        in_specs=[pl.BlockSpec((8, CBLK), lambda i, j: (i, j))],
        out_specs=pl.BlockSpec((8, 16), lambda i, j: (i, 0)),
        out_shape=jax.ShapeDtypeStruct((R, 16), jnp.float32),
        scratch_shapes=[pltpu.VMEM((8, 128), jnp.float32)],
    )(logits)


@functools.cache
def _build_stage_b():
    mesh = plsc.VectorSubcoreMesh(core_axis_name="c", subcore_axis_name="s")
    return functools.partial(
        pl.kernel,
        mesh=mesh,
        compiler_params=pltpu.CompilerParams(needs_layout_passes=False),
        out_type=[
            jax.ShapeDtypeStruct((R * CAP,), jnp.int32),
            jax.ShapeDtypeStruct((R * CAP,), jnp.float32),
            jax.ShapeDtypeStruct((R * 16,), jnp.int32),
        ],
        scratch_types=[
            pltpu.VMEM((4 * CHUNK,), jnp.float32),
            pltpu.VMEM((4 * CHUNK,), jnp.float32),
            pltpu.VMEM((4 * CAP,), jnp.int32),
            pltpu.VMEM((4 * CAP,), jnp.float32),
            pltpu.VMEM((64,), jnp.float32),
            pltpu.VMEM((64,), jnp.int32),
            pltpu.SemaphoreType.DMA,
            pltpu.SemaphoreType.DMA,
            pltpu.SemaphoreType.DMA,
        ],
    )(_stage_b_body)


def _stage_b_body(flat_hbm, t0_hbm, idx_hbm, val_hbm, cnt_hbm,
                  buf0, buf1, idx_v, val_v, t0_v, cnt_v, sem0, sem1, semg):
    wid = lax.axis_index("s") * 2 + lax.axis_index("c")
    row0 = wid * 4
    iota = lax.iota(jnp.int32, 16)

    pltpu.sync_copy(t0_hbm.at[pl.ds(pl.multiple_of(row0 * 16, 8), 64)], t0_v)
    tvecs = [t0_v[pl.ds(rl * 16, 16)] for rl in range(4)]
    idx_rows = [idx_v.at[pl.ds(rl * CAP, CAP)] for rl in range(4)]

    def zbody(z, _):
        idx_v[pl.ds(z * 16, 16)] = jnp.zeros((16,), jnp.int32)
        return 0
    lax.fori_loop(0, 4 * CAP // 16, zbody, 0)

    def issue(buf, sem, c):
        for rl in range(4):
            src = pl.multiple_of((row0 + rl) * V + c * CHUNK, 8)
            pltpu.async_copy(flat_hbm.at[pl.ds(src, CHUNK)],
                             buf.at[pl.ds(rl * CHUNK, CHUNK)], sem)

    def drain(buf, sem):
        for rl in range(4):
            pltpu.make_async_copy(flat_hbm.at[pl.ds(0, CHUNK)],
                                  buf.at[pl.ds(rl * CHUNK, CHUNK)], sem).wait()

    def scan_buf(buf, c, offs):
        @plsc.parallel_loop(0, CHUNK, 16, unroll=2, carry=offs)
        def inner(v, offs):
            new = []
            for rl in range(4):
                x = buf[pl.ds(rl * CHUNK + v, 16)]
                mask = x >= tvecs[rl]
                mi = jnp.where(mask, 1, 0).astype(jnp.int32)
                cum = plsc.cumsum(mi)
                pc = plsc.all_reduce_population_count(mask)
                idxv = iota + ((row0 + rl) * V + c * CHUNK + v)
                tgt = offs[rl] + cum
                smask = jnp.logical_and(mask, tgt < CAP)
                plsc.store_scatter(idx_rows[rl], [tgt], idxv, mask=smask)
                new.append(offs[rl] + pc)
            return tuple(new)

        return inner

    issue(buf0, sem0, 0)
    offs = (jnp.full((16,), -1, jnp.int32),) * 4

    def pair(c2, offs):
        c0 = c2 * 2
        issue(buf1, sem1, c0 + 1)
        drain(buf0, sem0)
        offs = scan_buf(buf0, c0, offs)

        @pl.when(c2 < NCHUNK // 2 - 1)
        def _():
            issue(buf0, sem0, c0 + 2)

        drain(buf1, sem1)
        offs = scan_buf(buf1, c0 + 1, offs)
        return offs

    offs = lax.fori_loop(0, NCHUNK // 2, pair, offs)

    for rl in range(4):
        cnt_v[pl.ds(rl * 16, 16)] = jnp.minimum(offs[rl] + 1, CAP)
        for g in range(CAP // 128):
            pltpu.async_copy(
                flat_hbm.at[idx_v.at[pl.ds(rl * CAP + g * 128, 128)]],
                val_v.at[pl.ds(rl * CAP + g * 128, 128)], semg).wait()

    pltpu.sync_copy(idx_v, idx_hbm.at[pl.ds(pl.multiple_of(row0 * CAP, 8),
                                            4 * CAP)])
    pltpu.sync_copy(val_v, val_hbm.at[pl.ds(pl.multiple_of(row0 * CAP, 8),
                                            4 * CAP)])
    pltpu.sync_copy(cnt_v, cnt_hbm.at[pl.ds(pl.multiple_of(row0 * 16, 8), 64)])


def _threefry_bits(x1u):
    k0 = jnp.uint32(0)
    k1 = jnp.uint32(42)
    k2 = k0 ^ k1 ^ jnp.uint32(0x1BD11BDA)
    ks = (k0, k1, k2)
    x0 = jnp.zeros_like(x1u) + k0
    x1 = x1u + k1
    rots = ((13, 15, 26, 6), (17, 29, 16, 24))
    for i in range(5):
        for rr in rots[i % 2]:
            x0 = x0 + x1
            x1 = (x1 << rr) | (x1 >> (32 - rr))
            x1 = x1 ^ x0
        x0 = x0 + ks[(i + 1) % 3]
        x1 = x1 + ks[(i + 2) % 3] + jnp.uint32(i + 1)
    return x0 ^ x1


def _stage_c_kernel(val_ref, idx_ref, cnt_ref, temp_ref, out_ref):
    vals = val_ref[...]
    idx = idx_ref[...]
    cnt = cnt_ref[...][:, 0:1]
    temp = temp_ref[...]

    colj = lax.broadcasted_iota(jnp.int32, (R, CAP), 1)
    rowi = lax.broadcasted_iota(jnp.int32, (R, CAP), 0)
    valid = colj < cnt

    l = vals / temp
    keys = jnp.where(valid, _monokey(l), INT_MIN)
    kth = _kth_largest_key(keys, K)
    keep = keys >= kth

    m = jnp.max(jnp.where(keep, l, NEG_INF), axis=1, keepdims=True)
    e = jnp.where(keep, jnp.exp(l - m), jnp.float32(0.0))
    s = jnp.sum(e, axis=1, keepdims=True)
    lp = jnp.log(e / s + jnp.float32(1e-30))

    bits = _threefry_bits(idx.astype(jnp.uint32))
    f = lax.bitcast_convert_type(
        (bits >> jnp.uint32(9)) | jnp.uint32(0x3F800000),
        jnp.float32) - jnp.float32(1.0)
    u = jnp.maximum(TINY, f + TINY)
    g = -jnp.log(-jnp.log(u))

    score = jnp.where(keep, lp + g, NEG_INF)
    best = jnp.max(score, axis=1, keepdims=True)
    wincol = jnp.min(jnp.where(score == best, colj, jnp.int32(2 ** 30)),
                     axis=1, keepdims=True)
    vocab_idx = idx - rowi * jnp.int32(V)
    token = jnp.sum(jnp.where(colj == wincol, vocab_idx, 0), axis=1,
                    keepdims=True)
    out_ref[...] = token


def _stage_c(vals, idx, cnts, temps):
    return pl.pallas_call(
        _stage_c_kernel,
        in_specs=[pl.BlockSpec((R, CAP), lambda: (0, 0)),
                  pl.BlockSpec((R, CAP), lambda: (0, 0)),
                  pl.BlockSpec((R, 16), lambda: (0, 0)),
                  pl.BlockSpec((R, 1), lambda: (0, 0))],
        out_specs=pl.BlockSpec((R, 1), lambda: (0, 0)),
        out_shape=jax.ShapeDtypeStruct((R, 1), jnp.int32),
    )(vals, idx, cnts, temps)


def kernel(logits, temperatures, top_k=50):
    del top_k
    logits = logits.astype(jnp.float32)
    t0 = _stage_a(logits)
    idx, vals, cnts = _build_stage_b()(logits.reshape(-1), t0.reshape(-1))
    tok = _stage_c(vals.reshape(R, CAP), idx.reshape(R, CAP),
                   cnts.reshape(R, 16),
                   temperatures.astype(jnp.float32).reshape(R, 1))
    return tok.reshape(R)


# speedup vs baseline: 2.2618x; 1.6859x over previous
import functools

import numpy as np
import jax
import jax.numpy as jnp
from jax import lax
from jax.experimental import pallas as pl
from jax.experimental.pallas import tpu as pltpu
from jax.experimental.pallas import tpu_sc as plsc

R = 128
V = 100000
K = 50
CAP = 512
CHUNK = 10000
NCHUNK = V // CHUNK
CBLK = 2048
NCB = (V + CBLK - 1) // CBLK
LAST_W = V - (NCB - 1) * CBLK
INT_MIN = np.int32(-2147483648)
NEG_INF = np.float32(-np.inf)
TINY = np.float32(1.1754943508222875e-38)


def _monokey(x):
    b = lax.bitcast_convert_type(x, jnp.int32)
    return jnp.where(b >= 0, b, b ^ jnp.int32(0x7FFFFFFF))


def _inv_monokey(k):
    b = jnp.where(k >= 0, k, k ^ jnp.int32(0x7FFFFFFF))
    return lax.bitcast_convert_type(b, jnp.float32)


def _kth_largest_key(keys, k):
    nonneg = jnp.sum(jnp.where(keys >= 0, 1, 0).astype(jnp.int32), axis=1,
                     keepdims=True)
    t = jnp.where(nonneg >= k, jnp.int32(0), INT_MIN)

    def body(i, t):
        cand = t | (jnp.int32(1) << (jnp.int32(30) - i))
        cnt = jnp.sum(jnp.where(keys >= cand, 1, 0).astype(jnp.int32),
                      axis=1, keepdims=True)
        return jnp.where(cnt >= k, cand, t)

    return lax.fori_loop(0, 31, body, t)


def _stage_a_kernel(x_ref, t0_ref):
    nfull = V // 128
    accs = [x_ref[:, s * 128:(s + 1) * 128] for s in range(8)]
    for s in range(8, nfull):
        accs[s % 8] = jnp.maximum(accs[s % 8], x_ref[:, s * 128:(s + 1) * 128])
    tail = x_ref[:, nfull * 128:V]
    accs[0] = jnp.maximum(
        accs[0],
        jnp.concatenate([tail, jnp.full((8, 96), NEG_INF, jnp.float32)], 1))
    m = accs[0]
    for s in range(1, 8):
        m = jnp.maximum(m, accs[s])

    keys = _monokey(m)
    t = _kth_largest_key(keys, K)
    t0 = _inv_monokey(t - 4)
    t0_ref[...] = jnp.broadcast_to(t0, (8, 16))


def _stage_a(logits):
    return pl.pallas_call(
        _stage_a_kernel,
        grid=(R // 8,),
        in_specs=[pl.BlockSpec((8, V), lambda i: (i, 0))],
        out_specs=pl.BlockSpec((8, 16), lambda i: (i, 0)),
        out_shape=jax.ShapeDtypeStruct((R, 16), jnp.float32),
    )(logits)


@functools.cache
def _build_stage_b():
    mesh = plsc.VectorSubcoreMesh(core_axis_name="c", subcore_axis_name="s")
    return functools.partial(
        pl.kernel,
        mesh=mesh,
        compiler_params=pltpu.CompilerParams(needs_layout_passes=False),
        out_type=[
            jax.ShapeDtypeStruct((R * CAP,), jnp.int32),
            jax.ShapeDtypeStruct((R * CAP,), jnp.float32),
            jax.ShapeDtypeStruct((R * 16,), jnp.int32),
        ],
        scratch_types=[
            pltpu.VMEM((4 * CHUNK,), jnp.float32),
            pltpu.VMEM((4 * CHUNK,), jnp.float32),
            pltpu.VMEM((4 * CAP,), jnp.int32),
            pltpu.VMEM((4 * CAP,), jnp.float32),
            pltpu.VMEM((64,), jnp.float32),
            pltpu.VMEM((64,), jnp.int32),
            pltpu.SemaphoreType.DMA,
            pltpu.SemaphoreType.DMA,
            pltpu.SemaphoreType.DMA,
        ],
    )(_stage_b_body)


def _stage_b_body(flat_hbm, t0_hbm, idx_hbm, val_hbm, cnt_hbm,
                  buf0, buf1, idx_v, val_v, t0_v, cnt_v, sem0, sem1, semg):
    wid = lax.axis_index("s") * 2 + lax.axis_index("c")
    row0 = wid * 4
    iota = lax.iota(jnp.int32, 16)

    pltpu.sync_copy(t0_hbm.at[pl.ds(pl.multiple_of(row0 * 16, 8), 64)], t0_v)
    tvecs = [t0_v[pl.ds(rl * 16, 16)] for rl in range(4)]
    idx_rows = [idx_v.at[pl.ds(rl * CAP, CAP)] for rl in range(4)]

    def zbody(z, _):
        idx_v[pl.ds(z * 16, 16)] = jnp.zeros((16,), jnp.int32)
        return 0
    lax.fori_loop(0, 4 * CAP // 16, zbody, 0)

    def issue(buf, sem, c):
        for rl in range(4):
            src = pl.multiple_of((row0 + rl) * V + c * CHUNK, 8)
            pltpu.async_copy(flat_hbm.at[pl.ds(src, CHUNK)],
                             buf.at[pl.ds(rl * CHUNK, CHUNK)], sem)

    def drain(buf, sem):
        for rl in range(4):
            pltpu.make_async_copy(flat_hbm.at[pl.ds(0, CHUNK)],
                                  buf.at[pl.ds(rl * CHUNK, CHUNK)], sem).wait()

    def scan_buf(buf, c, offs):
        @plsc.parallel_loop(0, CHUNK, 16, unroll=2, carry=offs)
        def inner(v, offs):
            new = []
            for rl in range(4):
                x = buf[pl.ds(rl * CHUNK + v, 16)]
                mask = x >= tvecs[rl]
                mi = jnp.where(mask, 1, 0).astype(jnp.int32)
                cum = plsc.cumsum(mi)
                pc = plsc.all_reduce_population_count(mask)
                idxv = iota + ((row0 + rl) * V + c * CHUNK + v)
                tgt = offs[rl] + cum
                smask = jnp.logical_and(mask, tgt < CAP)
                plsc.store_scatter(idx_rows[rl], [tgt], idxv, mask=smask)
                new.append(offs[rl] + pc)
            return tuple(new)

        return inner

    issue(buf0, sem0, 0)
    offs = (jnp.full((16,), -1, jnp.int32),) * 4

    def pair(c2, offs):
        c0 = c2 * 2
        issue(buf1, sem1, c0 + 1)
        drain(buf0, sem0)
        offs = scan_buf(buf0, c0, offs)

        @pl.when(c2 < NCHUNK // 2 - 1)
        def _():
            issue(buf0, sem0, c0 + 2)

        drain(buf1, sem1)
        offs = scan_buf(buf1, c0 + 1, offs)
        return offs

    offs = lax.fori_loop(0, NCHUNK // 2, pair, offs)

    for rl in range(4):
        cnt_v[pl.ds(rl * 16, 16)] = jnp.minimum(offs[rl] + 1, CAP)
        for g in range(CAP // 128):
            pltpu.async_copy(
                flat_hbm.at[idx_v.at[pl.ds(rl * CAP + g * 128, 128)]],
                val_v.at[pl.ds(rl * CAP + g * 128, 128)], semg).wait()

    pltpu.sync_copy(idx_v, idx_hbm.at[pl.ds(pl.multiple_of(row0 * CAP, 8),
                                            4 * CAP)])
    pltpu.sync_copy(val_v, val_hbm.at[pl.ds(pl.multiple_of(row0 * CAP, 8),
                                            4 * CAP)])
    pltpu.sync_copy(cnt_v, cnt_hbm.at[pl.ds(pl.multiple_of(row0 * 16, 8), 64)])


def _threefry_bits(x1u):
    k0 = jnp.uint32(0)
    k1 = jnp.uint32(42)
    k2 = k0 ^ k1 ^ jnp.uint32(0x1BD11BDA)
    ks = (k0, k1, k2)
    x0 = jnp.zeros_like(x1u) + k0
    x1 = x1u + k1
    rots = ((13, 15, 26, 6), (17, 29, 16, 24))
    for i in range(5):
        for rr in rots[i % 2]:
            x0 = x0 + x1
            x1 = (x1 << rr) | (x1 >> (32 - rr))
            x1 = x1 ^ x0
        x0 = x0 + ks[(i + 1) % 3]
        x1 = x1 + ks[(i + 2) % 3] + jnp.uint32(i + 1)
    return x0 ^ x1


def _stage_c_kernel(val_ref, idx_ref, cnt_ref, temp_ref, out_ref):
    vals = val_ref[...]
    idx = idx_ref[...]
    cnt = cnt_ref[...][:, 0:1]
    temp = temp_ref[...]

    colj = lax.broadcasted_iota(jnp.int32, (R, CAP), 1)
    rowi = lax.broadcasted_iota(jnp.int32, (R, CAP), 0)
    valid = colj < cnt

    l = vals / temp
    keys = jnp.where(valid, _monokey(l), INT_MIN)
    kth = _kth_largest_key(keys, K)
    keep = keys >= kth

    m = jnp.max(jnp.where(keep, l, NEG_INF), axis=1, keepdims=True)
    e = jnp.where(keep, jnp.exp(l - m), jnp.float32(0.0))
    s = jnp.sum(e, axis=1, keepdims=True)
    lp = jnp.log(e / s + jnp.float32(1e-30))

    bits = _threefry_bits(idx.astype(jnp.uint32))
    f = lax.bitcast_convert_type(
        (bits >> jnp.uint32(9)) | jnp.uint32(0x3F800000),
        jnp.float32) - jnp.float32(1.0)
    u = jnp.maximum(TINY, f + TINY)
    g = -jnp.log(-jnp.log(u))

    score = jnp.where(keep, lp + g, NEG_INF)
    best = jnp.max(score, axis=1, keepdims=True)
    wincol = jnp.min(jnp.where(score == best, colj, jnp.int32(2 ** 30)),
                     axis=1, keepdims=True)
    vocab_idx = idx - rowi * jnp.int32(V)
    token = jnp.sum(jnp.where(colj == wincol, vocab_idx, 0), axis=1,
                    keepdims=True)
    out_ref[...] = token


def _stage_c(vals, idx, cnts, temps):
    return pl.pallas_call(
        _stage_c_kernel,
        in_specs=[pl.BlockSpec((R, CAP), lambda: (0, 0)),
                  pl.BlockSpec((R, CAP), lambda: (0, 0)),
                  pl.BlockSpec((R, 16), lambda: (0, 0)),
                  pl.BlockSpec((R, 1), lambda: (0, 0))],
        out_specs=pl.BlockSpec((R, 1), lambda: (0, 0)),
        out_shape=jax.ShapeDtypeStruct((R, 1), jnp.int32),
    )(vals, idx, cnts, temps)


def kernel(logits, temperatures, top_k=50):
    del top_k
    logits = logits.astype(jnp.float32)
    t0 = _stage_a(logits)
    idx, vals, cnts = _build_stage_b()(logits.reshape(-1), t0.reshape(-1))
    tok = _stage_c(vals.reshape(R, CAP), idx.reshape(R, CAP),
                   cnts.reshape(R, 16),
                   temperatures.astype(jnp.float32).reshape(R, 1))
    return tok.reshape(R)
